# baseline (speedup 1.0000x reference)
import jax
import jax.numpy as jnp
import numpy as np
from jax.experimental import pallas as pl
from jax.experimental.pallas import tpu as pltpu

_B = 16
_A = 3
_C = 85
_H = 76
_W = 76
_HW = _H * _W
_K = _A * _C

_ANCHORS_ALL = [[10, 13], [16, 30], [33, 23], [30, 61], [62, 45],
                [59, 119], [116, 90], [156, 198], [373, 326]]
_MASK = [0, 1, 2]


def _decode_body(x_ref, smul_ref, emul_ref, ix_ref, iy_ref, mx_ref, my_ref,
                 o_ref):
    y = x_ref[0].T
    sig = jnp.tanh(y * 0.5) * 0.5 + 0.5
    e = jnp.exp(y)
    o_ref[0] = (sig * smul_ref[0] + e * emul_ref[0]
                + mx_ref[...] * ix_ref[0] + my_ref[...] * iy_ref[0])


def kernel(x, img_dim):
    x3 = x.reshape(_B, _K, _HW)
    stride = (img_dim[1] / _H).astype(jnp.float32)
    anchors = jnp.asarray(
        [_ANCHORS_ALL[i] for i in _MASK], dtype=jnp.float32) / stride

    c = np.arange(_K) % _C
    s_mul = jnp.where(jnp.asarray(c < 2), stride,
                      jnp.asarray((c >= 4).astype(np.float32)))[None]
    e_sel = np.zeros((_K, 2 * _A), np.float32)
    for a in range(_A):
        e_sel[a * _C + 2, 2 * a] = 1.0
        e_sel[a * _C + 3, 2 * a + 1] = 1.0
    e_mul = (jnp.asarray(e_sel) @ anchors.reshape(-1) * stride)[None]
    ix = jnp.asarray((c == 0).astype(np.float32))[None]
    iy = jnp.asarray((c == 1).astype(np.float32))[None]
    hw = np.arange(_HW)
    mx = (jnp.asarray((hw % _W).astype(np.float32)) * stride)[:, None]
    my = (jnp.asarray((hw // _W).astype(np.float32)) * stride)[:, None]

    lane_spec = pl.BlockSpec((1, _K), lambda b: (0, 0))
    row_spec = pl.BlockSpec((_HW, 1), lambda b: (0, 0))
    out = pl.pallas_call(
        _decode_body,
        grid=(_B,),
        in_specs=[
            pl.BlockSpec((1, _K, _HW), lambda b: (b, 0, 0)),
            lane_spec, lane_spec, lane_spec, lane_spec,
            row_spec, row_spec,
        ],
        out_specs=pl.BlockSpec((1, _HW, _K), lambda b: (b, 0, 0)),
        out_shape=jax.ShapeDtypeStruct((_B, _HW, _K), jnp.float32),
    )(x3, s_mul, e_mul, ix, iy, mx, my)
    return out.reshape(_B, _A * _HW, _C)
